# baseline (speedup 1.0000x reference)
import jax
import jax.numpy as jnp
from jax import lax
from jax.experimental import pallas as pl
from jax.experimental.pallas import tpu as pltpu

_NUM_EMBED = 1024
_COMMIT = 0.25
_LBLK = 1024


def _vq_body(x_ref, cb_ref, out_ref, idx_ref, loss_ref, cn_ref):
    first = jnp.logical_and(pl.program_id(0) == 0, pl.program_id(1) == 0)
    xb = jnp.transpose(x_ref[0], (1, 0))
    cb = cb_ref[...]

    @pl.when(first)
    def _precompute():
        cn_ref[...] = jnp.sum(cb * cb, axis=1)[None, :]

    cn = cn_ref[0]
    m2 = lax.dot_general(xb, cb + cb, (((1,), (1,)), ((), ())),
                         preferred_element_type=jnp.float32)
    a = jnp.sum(xb * xb, axis=1, keepdims=True)
    dist = (a - m2) + cn[None, :]
    dmin = jnp.min(dist, axis=1, keepdims=True)
    lane = lax.broadcasted_iota(jnp.int32, (1, _NUM_EMBED), 1).astype(jnp.float32)
    idxf = jnp.min(jnp.where(dist == dmin, lane, float(_NUM_EMBED)), axis=1)
    idx_ref[...] = idxf.astype(jnp.int32)[:, None]
    onehot = (lane == idxf[:, None]).astype(jnp.float32)
    quant = lax.dot_general(onehot, cb, (((1,), (0,)), ((), ())),
                            preferred_element_type=jnp.float32)
    diff = quant - xb
    out_ref[0] = jnp.transpose(xb + diff, (1, 0))

    @pl.when(first)
    def _init():
        loss_ref[0, 0] = 0.0

    loss_ref[0, 0] += jnp.sum(diff * diff)


def kernel(inputs, codebook):
    B, D, L = inputs.shape
    n_tok = B * L
    nj = L // _LBLK
    grid = (B, nj)
    out, idx, loss_sum = pl.pallas_call(
        _vq_body,
        grid=grid,
        in_specs=[
            pl.BlockSpec((1, D, _LBLK), lambda b, j: (b, 0, j)),
            pl.BlockSpec((_NUM_EMBED, D), lambda b, j: (0, 0)),
        ],
        out_specs=[
            pl.BlockSpec((1, D, _LBLK), lambda b, j: (b, 0, j)),
            pl.BlockSpec((_LBLK, 1), lambda b, j: (b * nj + j, 0)),
            pl.BlockSpec((1, 1), lambda b, j: (0, 0), memory_space=pltpu.SMEM),
        ],
        out_shape=[
            jax.ShapeDtypeStruct((B, D, L), jnp.float32),
            jax.ShapeDtypeStruct((n_tok, 1), jnp.int32),
            jax.ShapeDtypeStruct((1, 1), jnp.float32),
        ],
        scratch_shapes=[pltpu.VMEM((1, _NUM_EMBED), jnp.float32)],
    )(inputs, codebook)
    s = loss_sum[0, 0] / (n_tok * D)
    loss = s + _COMMIT * s
    return out, loss, idx.reshape(B, L)


# speedup vs baseline: 1.5158x; 1.5158x over previous
import jax
import jax.numpy as jnp
from jax import lax
from jax.experimental import pallas as pl
from jax.experimental.pallas import tpu as pltpu

_K = 1024
_COMMIT = 0.25
_LBLK = 1024


def _vq_body(x_ref, cb_ref, out_ref, idx_ref, loss_ref, cnbc_ref, code_ref):
    first = jnp.logical_and(pl.program_id(0) == 0, pl.program_id(1) == 0)
    xT = x_ref[0]
    cb = cb_ref[...]

    @pl.when(first)
    def _precompute():
        cn = jnp.sum(cb * cb, axis=1, keepdims=True)
        cnbc_ref[...] = jnp.broadcast_to(cn, (_K, _LBLK))
        code_ref[...] = lax.broadcasted_iota(
            jnp.int32, (_K, _LBLK), 0).astype(jnp.float32)
        loss_ref[0, 0] = 0.0

    xb = jnp.transpose(xT, (1, 0))
    a = jnp.sum(xb * xb, axis=1, keepdims=True)
    aT = jnp.transpose(a, (1, 0))
    m2T = lax.dot_general(cb + cb, xT, (((1,), (0,)), ((), ())),
                          preferred_element_type=jnp.float32)
    dist = (aT - m2T) + cnbc_ref[...]
    dminT = jnp.min(dist, axis=0, keepdims=True)
    code = code_ref[...]
    cand = jnp.where(dist == dminT, code, float(_K))
    idxfT = jnp.min(cand, axis=0, keepdims=True)
    idx_ref[0] = idxfT.astype(jnp.int32)
    onehot = (code == idxfT).astype(jnp.float32)
    quantT = lax.dot_general(cb, onehot, (((0,), (0,)), ((), ())),
                             preferred_element_type=jnp.float32)
    diff = quantT - xT
    out_ref[0] = xT + diff
    loss_ref[0, 0] += jnp.sum(diff * diff)


def kernel(inputs, codebook):
    B, D, L = inputs.shape
    nj = L // _LBLK
    grid = (B, nj)
    out, idx, loss_sum = pl.pallas_call(
        _vq_body,
        grid=grid,
        in_specs=[
            pl.BlockSpec((1, D, _LBLK), lambda b, j: (b, 0, j)),
            pl.BlockSpec((_K, D), lambda b, j: (0, 0)),
        ],
        out_specs=[
            pl.BlockSpec((1, D, _LBLK), lambda b, j: (b, 0, j)),
            pl.BlockSpec((1, 1, _LBLK), lambda b, j: (b, 0, j)),
            pl.BlockSpec((1, 1), lambda b, j: (0, 0), memory_space=pltpu.SMEM),
        ],
        out_shape=[
            jax.ShapeDtypeStruct((B, D, L), jnp.float32),
            jax.ShapeDtypeStruct((B, 1, L), jnp.int32),
            jax.ShapeDtypeStruct((1, 1), jnp.float32),
        ],
        scratch_shapes=[
            pltpu.VMEM((_K, _LBLK), jnp.float32),
            pltpu.VMEM((_K, _LBLK), jnp.float32),
        ],
    )(inputs, codebook)
    s = loss_sum[0, 0] / (B * L * D)
    loss = s + _COMMIT * s
    return out, loss, idx.reshape(B, L)


# speedup vs baseline: 1.6068x; 1.0600x over previous
import jax
import jax.numpy as jnp
from jax import lax
from jax.experimental import pallas as pl
from jax.experimental.pallas import tpu as pltpu

_K = 1024
_COMMIT = 0.25
_LBLK = 1024


def _vq_body(x_ref, cb_ref, out_ref, idx_ref, loss_ref, cnbc_ref, code_ref):
    first = jnp.logical_and(pl.program_id(0) == 0, pl.program_id(1) == 0)
    xT = x_ref[0]
    cb = cb_ref[...]

    @pl.when(first)
    def _precompute():
        cn = jnp.sum(cb * cb, axis=1, keepdims=True)
        cnbc_ref[...] = jnp.broadcast_to(cn, (_K, _LBLK))
        code_ref[...] = lax.broadcasted_iota(
            jnp.int32, (_K, _LBLK), 0).astype(jnp.float32)
        loss_ref[0, 0] = 0.0

    aT = jnp.sum(xT * xT, axis=0, keepdims=True)
    m2T = lax.dot_general(cb + cb, xT, (((1,), (0,)), ((), ())),
                          preferred_element_type=jnp.float32)
    dist = (aT - m2T) + cnbc_ref[...]
    dminT = jnp.min(dist, axis=0, keepdims=True)
    code = code_ref[...]
    cand = jnp.where(dist == dminT, code, float(_K))
    idxfT = jnp.min(cand, axis=0, keepdims=True)
    idx_ref[0] = idxfT.astype(jnp.int32)
    onehot = (code == idxfT).astype(jnp.float32)
    quantT = lax.dot_general(cb, onehot, (((0,), (0,)), ((), ())),
                             preferred_element_type=jnp.float32)
    diff = quantT - xT
    out_ref[0] = xT + diff
    loss_ref[0, 0] += jnp.sum(diff * diff)


def kernel(inputs, codebook):
    B, D, L = inputs.shape
    nj = L // _LBLK
    grid = (B, nj)
    out, idx, loss_sum = pl.pallas_call(
        _vq_body,
        grid=grid,
        in_specs=[
            pl.BlockSpec((1, D, _LBLK), lambda b, j: (b, 0, j)),
            pl.BlockSpec((_K, D), lambda b, j: (0, 0)),
        ],
        out_specs=[
            pl.BlockSpec((1, D, _LBLK), lambda b, j: (b, 0, j)),
            pl.BlockSpec((1, 1, _LBLK), lambda b, j: (b, 0, j)),
            pl.BlockSpec((1, 1), lambda b, j: (0, 0), memory_space=pltpu.SMEM),
        ],
        out_shape=[
            jax.ShapeDtypeStruct((B, D, L), jnp.float32),
            jax.ShapeDtypeStruct((B, 1, L), jnp.int32),
            jax.ShapeDtypeStruct((1, 1), jnp.float32),
        ],
        scratch_shapes=[
            pltpu.VMEM((_K, _LBLK), jnp.float32),
            pltpu.VMEM((_K, _LBLK), jnp.float32),
        ],
    )(inputs, codebook)
    s = loss_sum[0, 0] / (B * L * D)
    loss = s + _COMMIT * s
    return out, loss, idx.reshape(B, L)
